# baseline (speedup 1.0000x reference)
import functools

import jax
import jax.numpy as jnp
from jax import lax
from jax.experimental import pallas as pl
from jax.experimental.pallas import tpu as pltpu
from jax.experimental.pallas import tpu_sc as plsc

_NUM_CAMERAS = 100000
_BATCH = 16384

_NC = 2
_NS = 16
_NW = _NC * _NS
_B_PER_W = _BATCH // _NW
_CHUNK = 128
_NCHUNK = _B_PER_W // _CHUNK


@functools.partial(
    pl.kernel,
    mesh=plsc.VectorSubcoreMesh(core_axis_name="c", subcore_axis_name="s"),
    out_type=jax.ShapeDtypeStruct((_NW, _NCHUNK, _CHUNK), jnp.float32),
    scratch_types=[
        pltpu.VMEM((_NCHUNK, _CHUNK), jnp.int32),
        pltpu.VMEM((_NCHUNK, _CHUNK), jnp.float32),
        pltpu.SemaphoreType.DMA,
        pltpu.SemaphoreType.DMA,
        pltpu.SemaphoreType.DMA,
        pltpu.SemaphoreType.DMA,
        pltpu.SemaphoreType.DMA,
    ],
)
def _sc_gather(idx_hbm, table_hbm, out_hbm, idx_v, vals_v,
               sg0, sg1, sg2, sg3, so):
    sg = (sg0, sg1, sg2, sg3)
    wid = lax.axis_index("s") * _NC + lax.axis_index("c")
    pltpu.sync_copy(idx_hbm.at[wid], idx_v)
    hg = [
        pltpu.async_copy(table_hbm.at[idx_v.at[j]], vals_v.at[j], sg[j])
        for j in range(_NCHUNK)
    ]
    oblk = out_hbm.at[wid]
    ho = []
    for j in range(_NCHUNK):
        hg[j].wait()
        ho.append(pltpu.async_copy(vals_v.at[j], oblk.at[j], so))
    for c in ho:
        c.wait()


def kernel(indices, adjustment):
    idx = indices.astype(jnp.int32).reshape(_NW, _NCHUNK, _CHUNK)
    out = _sc_gather(idx, adjustment)
    return out.reshape(_BATCH)


# speedup vs baseline: 1.0655x; 1.0655x over previous
import functools

import jax
import jax.numpy as jnp
from jax import lax
from jax.experimental import pallas as pl
from jax.experimental.pallas import tpu as pltpu
from jax.experimental.pallas import tpu_sc as plsc

_NUM_CAMERAS = 100000
_BATCH = 16384

_NC = 1
_NS = 16
_NW = _NC * _NS
_B_PER_W = _BATCH // _NW
_CHUNK = 128
_NCHUNK = _B_PER_W // _CHUNK


@functools.partial(
    pl.kernel,
    mesh=plsc.VectorSubcoreMesh(
        core_axis_name="c", subcore_axis_name="s", num_cores=_NC
    ),
    out_type=jax.ShapeDtypeStruct((_NW, _NCHUNK, _CHUNK), jnp.float32),
    scratch_types=[
        pltpu.VMEM((_NCHUNK, _CHUNK), jnp.int32),
        pltpu.VMEM((_NCHUNK, _CHUNK), jnp.float32),
    ]
    + [pltpu.SemaphoreType.DMA] * (_NCHUNK + 1),
)
def _sc_gather(idx_hbm, table_hbm, out_hbm, idx_v, vals_v, *sems):
    sg, so = sems[:_NCHUNK], sems[_NCHUNK]
    wid = lax.axis_index("s") * _NC + lax.axis_index("c")
    pltpu.sync_copy(idx_hbm.at[wid], idx_v)
    hg = [
        pltpu.async_copy(table_hbm.at[idx_v.at[j]], vals_v.at[j], sg[j])
        for j in range(_NCHUNK)
    ]
    oblk = out_hbm.at[wid]
    ho = []
    for j in range(_NCHUNK):
        hg[j].wait()
        ho.append(pltpu.async_copy(vals_v.at[j], oblk.at[j], so))
    for c in ho:
        c.wait()


def kernel(indices, adjustment):
    idx = indices.astype(jnp.int32).reshape(_NW, _NCHUNK, _CHUNK)
    out = _sc_gather(idx, adjustment)
    return out.reshape(_BATCH)
